# baseline (speedup 1.0000x reference)
import functools

import jax
import jax.numpy as jnp
from jax import lax
from jax.experimental import pallas as pl
from jax.experimental.pallas import tpu as pltpu
from jax.experimental.pallas import tpu_sc as plsc

V = 1000000
D = 32
B = 16384
GAMMA = 0.5
NW = 32
BPW = B // NW
NBLK = BPW // 16


def _vrsqrt(x):
    i = plsc.bitcast(x, jnp.int32)
    i = jnp.int32(0x5F3759DF) - (i >> 1)
    y = plsc.bitcast(i, jnp.float32)
    half_x = 0.5 * x
    for _ in range(3):
        y = y * (1.5 - half_x * y * y)
    return y


def _sc_kernel(idx_hbm, pt_hbm, ct_hbm, wb_hbm,
               rating_hbm, scale_hbm, dirt_hbm,
               idx_v, tiles_p, tiles_c, tiles_p2, tiles_c2, blk_p, blk_c,
               wb_v, scale_v, rating_v, dirt_v, sem, sem2):
    wid = lax.axis_index("s") * 2 + lax.axis_index("c")
    base = wid * BPW

    pltpu.sync_copy(idx_hbm.at[wid], idx_v)
    pltpu.sync_copy(wb_hbm, wb_v)

    d_ids = lax.iota(jnp.int32, 16)
    wlow = wb_v[0, pl.ds(0, 16)]
    whigh = wb_v[0, pl.ds(16, 16)]
    bias = jnp.full((16,), 1.0, jnp.float32) * wb_v[1, pl.ds(0, 16)][0]

    def blk_body(blk, carry):
        b16 = blk * 16
        vec = idx_v[pl.ds(b16, 16)]
        jvec = (vec >> 7) * 128
        cvec = vec & 127

        def fire(q, t_p, t_c, s):
            cps = []
            for l4 in range(4):
                j = pl.multiple_of(jvec[q * 4 + l4], 128)
                cps.append(pltpu.async_copy(
                    pt_hbm.at[:, pl.ds(j, 128)], t_p.at[l4], s))
                cps.append(pltpu.async_copy(
                    ct_hbm.at[:, pl.ds(j, 128)], t_c.at[l4], s))
            return cps

        def extract(q, t_p, t_c):
            for l4 in range(4):
                l = q * 4 + l4
                cl = jnp.full((16,), 0, jnp.int32) + cvec[l]
                lsplat = jnp.full((16,), l, jnp.int32)
                l4s = jnp.full((16,), l4, jnp.int32)
                for half in range(2):
                    rows = d_ids + (half * 16)
                    pcol = plsc.load_gather(t_p, [l4s, rows, cl])
                    ccol = plsc.load_gather(t_c, [l4s, rows, cl])
                    plsc.store_scatter(blk_p, [rows, lsplat], pcol)
                    plsc.store_scatter(blk_c, [rows, lsplat], ccol)

        sets = ((tiles_p, tiles_c, sem), (tiles_p2, tiles_c2, sem2))
        pend = fire(0, *sets[0])
        for q in range(4):
            nxt = fire(q + 1, *sets[(q + 1) % 2]) if q < 3 else []
            for cp in pend:
                cp.wait()
            extract(q, sets[q % 2][0], sets[q % 2][1])
            pend = nxt

        acc_c2 = jnp.full((16,), 0.0, jnp.float32)
        acc_v2 = jnp.full((16,), 0.0, jnp.float32)
        for d in range(D):
            pc = blk_p[d, pl.ds(0, 16)]
            cc = blk_c[d, pl.ds(0, 16)]
            vc = GAMMA * pc + cc
            acc_c2 = acc_c2 + cc * cc
            acc_v2 = acc_v2 + vc * vc
            dirt_v[d, pl.ds(b16, 16)] = vc
        scale = acc_c2 * _vrsqrt(jnp.maximum(acc_c2, jnp.float32(1e-30)))
        rinv = _vrsqrt(jnp.maximum(acc_v2, jnp.float32(1e-24)))
        logit = jnp.full((16,), 0.0, jnp.float32)
        for d in range(D):
            dirc = dirt_v[d, pl.ds(b16, 16)] * rinv
            dirt_v[d, pl.ds(b16, 16)] = dirc
            wsrc = wlow if d < 16 else whigh
            wd = jnp.full((16,), 1.0, jnp.float32) * wsrc[d % 16]
            logit = logit + dirc * wd
        logit = scale * logit + bias
        rating = 1.0 / (1.0 + jnp.exp(-logit))
        scale_v[pl.ds(b16, 16)] = scale
        rating_v[pl.ds(b16, 16)] = rating
        return carry

    lax.fori_loop(0, NBLK, blk_body, 0)

    pltpu.sync_copy(rating_v, rating_hbm.at[pl.ds(base, BPW)])
    pltpu.sync_copy(scale_v, scale_hbm.at[pl.ds(base, BPW)])
    pltpu.sync_copy(dirt_v, dirt_hbm.at[:, pl.ds(base, BPW)])


@jax.jit
def _run(idx2, Pt, Ct, wb):
    mesh = plsc.VectorSubcoreMesh(core_axis_name="c", subcore_axis_name="s")
    k = functools.partial(
        pl.kernel, mesh=mesh,
        compiler_params=pltpu.CompilerParams(
            needs_layout_passes=False, use_tc_tiling_on_sc=True),
        out_type=(
            jax.ShapeDtypeStruct((B,), jnp.float32),
            jax.ShapeDtypeStruct((B,), jnp.float32),
            jax.ShapeDtypeStruct((D, B), jnp.float32),
        ),
        scratch_types=[
            pltpu.VMEM((BPW,), jnp.int32),
            pltpu.VMEM((4, D, 128), jnp.float32),
            pltpu.VMEM((4, D, 128), jnp.float32),
            pltpu.VMEM((4, D, 128), jnp.float32),
            pltpu.VMEM((4, D, 128), jnp.float32),
            pltpu.VMEM((D, 128), jnp.float32),
            pltpu.VMEM((D, 128), jnp.float32),
            pltpu.VMEM((8, 128), jnp.float32),
            pltpu.VMEM((BPW,), jnp.float32),
            pltpu.VMEM((BPW,), jnp.float32),
            pltpu.VMEM((D, BPW), jnp.float32),
            pltpu.SemaphoreType.DMA,
            pltpu.SemaphoreType.DMA,
        ],
    )(_sc_kernel)
    return k(idx2, Pt, Ct, wb)


def kernel(item_indices, P, C, W, b):
    idx2 = item_indices.reshape(NW, BPW).astype(jnp.int32)
    wb = jnp.zeros((8, 128), jnp.float32)
    wb = wb.at[0, :D].set(W.reshape(D))
    wb = wb.at[1, 0].set(b[0])
    rating, scale, dirt = _run(idx2, P.T, C.T, wb)
    return (rating.reshape(B, 1), scale.reshape(B, 1), dirt.T)
